# Initial kernel scaffold; baseline (speedup 1.0000x reference)
import jax
import jax.numpy as jnp
from jax.experimental import pallas as pl


def kernel(h, pos, edge_index, e, Wm1, bm1, Wm2, bm2, Wc1, bc1, Wc2, bc2, Wn1, bn1, Wn2, bn2, gamma, beta):
    raise NotImplementedError("write your pallas kernel here")


# speedup vs baseline: 3.8780x; 3.8780x over previous
import functools

import jax
import jax.numpy as jnp
from jax import lax
from jax.experimental import pallas as pl
from jax.experimental.pallas import tpu as pltpu
from jax.experimental.pallas import tpu_sc as plsc

N = 10000
E = 320000
D = 128
EPS = 1e-08

NC = 2
NS = 16
NW = NC * NS
EPW = E // NW
CH = 80
NCH = EPW // CH
NZCH = N // CH
P16 = 16

_mesh = plsc.VectorSubcoreMesh(
    core_axis_name="c", subcore_axis_name="s", num_cores=NC, num_subcores=NS
)


VPC = CH // 16


@functools.partial(
    pl.kernel,
    out_type=(
        jax.ShapeDtypeStruct((E, D), jnp.float32),
        jax.ShapeDtypeStruct((E, D), jnp.float32),
        jax.ShapeDtypeStruct((E, P16), jnp.float32),
    ),
    mesh=_mesh,
    scratch_types=[
        pltpu.VMEM((NCH, CH), jnp.int32),
        pltpu.VMEM((NCH, CH), jnp.int32),
        pltpu.VMEM((CH, D), jnp.float32),
        pltpu.VMEM((CH, D), jnp.float32),
        pltpu.VMEM((3 * N,), jnp.float32),
        pltpu.VMEM((CH, P16), jnp.float32),
        pltpu.SemaphoreType.DMA,
    ],
    compiler_params=pltpu.CompilerParams(needs_layout_passes=False),
)
def _sc_gather(ridx_hbm, sidx_hbm, h_hbm, posf_hbm,
               hr_hbm, hs_hbm, rd_hbm,
               ridx_v, sidx_v, hbr, hbs, posf, rbuf, sem):
    c = lax.axis_index("c")
    s = lax.axis_index("s")
    w = c * NS + s
    pltpu.sync_copy(ridx_hbm.at[w], ridx_v)
    pltpu.sync_copy(sidx_hbm.at[w], sidx_v)
    pltpu.sync_copy(posf_hbm, posf)
    base = w * EPW

    def body(j, carry):
        ir = ridx_v.at[j]
        isx = sidx_v.at[j]
        cp1 = pltpu.async_copy(h_hbm.at[ir], hbr, sem)
        cp2 = pltpu.async_copy(h_hbm.at[isx], hbs, sem)
        for v in range(VPC):
            irv = ridx_v.at[j][pl.ds(v * 16, 16)]
            isv = sidx_v.at[j][pl.ds(v * 16, 16)]
            ir3 = irv * 3
            is3 = isv * 3
            rx = plsc.load_gather(posf, [ir3]) - plsc.load_gather(posf, [is3])
            ry = (plsc.load_gather(posf, [ir3 + 1])
                  - plsc.load_gather(posf, [is3 + 1]))
            rz = (plsc.load_gather(posf, [ir3 + 2])
                  - plsc.load_gather(posf, [is3 + 2]))
            dsq = rx * rx + ry * ry + rz * rz
            eid = lax.iota(jnp.int32, 16) + v * 16
            c0 = jnp.zeros((16,), jnp.int32)
            plsc.store_scatter(rbuf, [eid, c0], rx)
            plsc.store_scatter(rbuf, [eid, c0 + 1], ry)
            plsc.store_scatter(rbuf, [eid, c0 + 2], rz)
            plsc.store_scatter(rbuf, [eid, c0 + 3], dsq)
        cp1.wait()
        cp2.wait()
        o = pl.ds(base + j * CH, CH)
        pltpu.sync_copy(hbr, hr_hbm.at[o])
        pltpu.sync_copy(hbs, hs_hbm.at[o])
        pltpu.sync_copy(rbuf, rd_hbm.at[o])
        return carry

    lax.fori_loop(0, NCH, body, 0)


@functools.partial(
    pl.kernel,
    out_type=(
        jax.ShapeDtypeStruct((NC, N, D), jnp.float32),
        jax.ShapeDtypeStruct((NC, N, D), jnp.float32),
    ),
    mesh=_mesh,
    scratch_types=[
        pltpu.VMEM((NCH, CH), jnp.int32),
        pltpu.VMEM((CH, D), jnp.float32),
        pltpu.VMEM_SHARED((N, D), jnp.float32),
        pltpu.SemaphoreType.DMA,
    ],
    compiler_params=pltpu.CompilerParams(needs_layout_passes=False),
)
def _sc_scatter(ridx_hbm, mij_hbm, cd_hbm, z128_hbm,
                macc_hbm, pacc_hbm,
                ridx_v, mbuf, macc_s, sem):
    c = lax.axis_index("c")
    s = lax.axis_index("s")
    w = c * NS + s
    pltpu.sync_copy(ridx_hbm.at[w], ridx_v)
    base = w * EPW

    for src_hbm, out_hbm in ((mij_hbm, macc_hbm), (cd_hbm, pacc_hbm)):
        pltpu.sync_copy(z128_hbm, mbuf)
        for t in range(8):
            k = s * 8 + t

            @pl.when(k < NZCH)
            def _():
                pltpu.sync_copy(mbuf, macc_s.at[pl.ds(k * CH, CH)])

        plsc.subcore_barrier()

        def body(j, carry):
            o = pl.ds(base + j * CH, CH)
            pltpu.async_copy(src_hbm.at[o], mbuf, sem).wait()
            pltpu.sync_copy(mbuf, macc_s.at[ridx_v.at[j]], add=True)
            return carry

        lax.fori_loop(0, NCH, body, 0)
        plsc.subcore_barrier()
        for t in range(8):
            k = s * 8 + t

            @pl.when(k < NZCH)
            def _():
                o = pl.ds(k * CH, CH)
                pltpu.sync_copy(macc_s.at[o], mbuf)
                pltpu.sync_copy(mbuf, out_hbm.at[c, o])

        plsc.subcore_barrier()


BE = 1600
GE = E // BE


def _edge_body(hr, hs, e, rd,
               Wr, Ws, We, wd, bm1, Wm2, bm2, Wc1, bc1, Wc2p, bc2p,
               mij_o, cd_o):
    rd_v = rd[...]
    dsq = rd_v[:, 3:4]
    dij = jnp.sqrt(dsq + EPS)
    hid = (jnp.dot(hr[...], Wr[...], preferred_element_type=jnp.float32)
           + jnp.dot(hs[...], Ws[...], preferred_element_type=jnp.float32)
           + jnp.dot(e[...], We[...], preferred_element_type=jnp.float32)
           + dij * wd[...] + bm1[...])
    hid = hid * jax.nn.sigmoid(hid)
    m = jnp.dot(hid, Wm2[...], preferred_element_type=jnp.float32) + bm2[...]
    mij_o[...] = m
    hc = jnp.dot(m, Wc1[...], preferred_element_type=jnp.float32) + bc1[...]
    hc = hc * jax.nn.sigmoid(hc)
    s128 = jnp.dot(hc, Wc2p[...], preferred_element_type=jnp.float32) + bc2p[...]
    lane = lax.broadcasted_iota(jnp.int32, (BE, D), 1)
    rd128 = jnp.concatenate(
        [rd_v, jnp.zeros((BE, D - P16), jnp.float32)], axis=1)
    rd128 = jnp.where(lane < 3, rd128, 0.0)
    cd_o[...] = rd128 * (s128 / (dij + EPS))


def _edge_tc(hr, hs, e, rd, Wr, Ws, We, wd, bm1, Wm2, bm2, Wc1, bc1,
             Wc2p, bc2p):
    full = lambda shape: pl.BlockSpec(shape, lambda i: (0, 0))
    blk = lambda width: pl.BlockSpec((BE, width), lambda i: (i, 0))
    return pl.pallas_call(
        _edge_body,
        grid=(GE,),
        in_specs=[
            blk(D), blk(D), blk(D), blk(P16),
            full((D, 2 * D)), full((D, 2 * D)), full((D, 2 * D)),
            full((1, 2 * D)), full((1, 2 * D)),
            full((2 * D, D)), full((1, D)),
            full((D, D)), full((1, D)),
            full((D, D)), full((1, D)),
        ],
        out_specs=[blk(D), blk(D)],
        out_shape=[
            jax.ShapeDtypeStruct((E, D), jnp.float32),
            jax.ShapeDtypeStruct((E, D), jnp.float32),
        ],
    )(hr, hs, e, rd, Wr, Ws, We, wd, bm1, Wm2, bm2, Wc1, bc1, Wc2p, bc2p)


BN = 2000
GN = N // BN


def _node_body(h, m0, m1, p128, d0, d1,
               Wn1h, Wn1m, bn1, Wn2, bn2, g, b,
               ho_o, po_o):
    m = m0[...] + m1[...]
    hid = (jnp.dot(h[...], Wn1h[...], preferred_element_type=jnp.float32)
           + jnp.dot(m, Wn1m[...], preferred_element_type=jnp.float32)
           + bn1[...])
    hid = hid * jax.nn.sigmoid(hid)
    ho = h[...] + jnp.dot(hid, Wn2[...], preferred_element_type=jnp.float32) + bn2[...]
    mu = jnp.mean(ho, axis=1, keepdims=True)
    zc = ho - mu
    var = jnp.mean(zc * zc, axis=1, keepdims=True)
    ho_o[...] = zc / jnp.sqrt(var + 1e-05) * g[...] + b[...]
    po_o[...] = p128[...] + d0[...] + d1[...]


def _node_tc(h, m0, m1, p128, d0, d1, Wn1h, Wn1m, bn1, Wn2, bn2, g, b):
    full = lambda shape: pl.BlockSpec(shape, lambda i: (0, 0))
    blk = lambda width: pl.BlockSpec((BN, width), lambda i: (i, 0))
    return pl.pallas_call(
        _node_body,
        grid=(GN,),
        in_specs=[
            blk(D), blk(D), blk(D), blk(D), blk(D), blk(D),
            full((D, 2 * D)), full((D, 2 * D)), full((1, 2 * D)),
            full((2 * D, D)), full((1, D)),
            full((1, D)), full((1, D)),
        ],
        out_specs=[blk(D), blk(D)],
        out_shape=[
            jax.ShapeDtypeStruct((N, D), jnp.float32),
            jax.ShapeDtypeStruct((N, D), jnp.float32),
        ],
    )(h, m0, m1, p128, d0, d1, Wn1h, Wn1m, bn1, Wn2, bn2, g, b)


def kernel(h, pos, edge_index, e, Wm1, bm1, Wm2, bm2, Wc1, bc1, Wc2, bc2,
           Wn1, bn1, Wn2, bn2, gamma, beta):
    send = edge_index[0].astype(jnp.int32).reshape(NW, NCH, CH)
    recv = edge_index[1].astype(jnp.int32).reshape(NW, NCH, CH)
    pos128 = jnp.pad(pos, ((0, 0), (0, D - 3)))

    hr, hs, rd = _sc_gather(recv, send, h, jnp.ravel(pos))

    Wr = Wm1[:D]
    Ws = Wm1[D:2 * D]
    We = Wm1[2 * D:3 * D]
    wd = Wm1[3 * D:3 * D + 1]
    Wc2p = jnp.tile(Wc2, (1, D))
    bc2p = jnp.tile(bc2.reshape(1, 1), (1, D))

    mij, cd = _edge_tc(hr, hs, e, rd, Wr, Ws, We, wd,
                       bm1.reshape(1, -1), Wm2, bm2.reshape(1, -1),
                       Wc1, bc1.reshape(1, -1), Wc2p, bc2p)

    z128 = jnp.zeros((CH, D), jnp.float32)
    macc, pacc = _sc_scatter(recv, mij, cd, z128)

    h_out, po128 = _node_tc(h, macc[0], macc[1], pos128, pacc[0], pacc[1],
                            Wn1[:D], Wn1[D:], bn1.reshape(1, -1),
                            Wn2, bn2.reshape(1, -1),
                            gamma.reshape(1, -1), beta.reshape(1, -1))
    return h_out, po128[:, :3]


# speedup vs baseline: 4.8214x; 1.2433x over previous
import functools

import jax
import jax.numpy as jnp
from jax import lax
from jax.experimental import pallas as pl
from jax.experimental.pallas import tpu as pltpu
from jax.experimental.pallas import tpu_sc as plsc

N = 10000
E = 320000
D = 128
EPS = 1e-08

NC = 2
NS = 16
NW = NC * NS
EPW = E // NW
CH = 80
NCH = EPW // CH
NZCH = N // CH
P16 = 16

_mesh = plsc.VectorSubcoreMesh(
    core_axis_name="c", subcore_axis_name="s", num_cores=NC, num_subcores=NS
)


VPC = CH // 16


@functools.partial(
    pl.kernel,
    out_type=(
        jax.ShapeDtypeStruct((E, D), jnp.float32),
        jax.ShapeDtypeStruct((E, D), jnp.float32),
        jax.ShapeDtypeStruct((E, P16), jnp.float32),
    ),
    mesh=_mesh,
    scratch_types=[
        pltpu.VMEM((NCH, CH), jnp.int32),
        pltpu.VMEM((NCH, CH), jnp.int32),
        pltpu.VMEM((CH, D), jnp.float32),
        pltpu.VMEM((CH, D), jnp.float32),
        pltpu.VMEM((CH, D), jnp.float32),
        pltpu.VMEM((CH, D), jnp.float32),
        pltpu.VMEM((3 * N,), jnp.float32),
        pltpu.VMEM((CH, P16), jnp.float32),
        pltpu.SemaphoreType.DMA,
        pltpu.SemaphoreType.DMA,
    ],
    compiler_params=pltpu.CompilerParams(needs_layout_passes=False),
)
def _sc_gather(ridx_hbm, sidx_hbm, h_hbm, posf_hbm,
               hr_hbm, hs_hbm, rd_hbm,
               ridx_v, sidx_v, hbr0, hbs0, hbr1, hbs1, posf, rbuf,
               sem0, sem1):
    c = lax.axis_index("c")
    s = lax.axis_index("s")
    w = c * NS + s
    pltpu.sync_copy(ridx_hbm.at[w], ridx_v)
    pltpu.sync_copy(sidx_hbm.at[w], sidx_v)
    pltpu.sync_copy(posf_hbm, posf)
    base = w * EPW

    pltpu.async_copy(h_hbm.at[ridx_v.at[0]], hbr0, sem0)
    pltpu.async_copy(h_hbm.at[sidx_v.at[0]], hbs0, sem0)

    def compute_rd(j):
        for v in range(VPC):
            irv = ridx_v.at[j][pl.ds(v * 16, 16)]
            isv = sidx_v.at[j][pl.ds(v * 16, 16)]
            ir3 = irv * 3
            is3 = isv * 3
            rx = plsc.load_gather(posf, [ir3]) - plsc.load_gather(posf, [is3])
            ry = (plsc.load_gather(posf, [ir3 + 1])
                  - plsc.load_gather(posf, [is3 + 1]))
            rz = (plsc.load_gather(posf, [ir3 + 2])
                  - plsc.load_gather(posf, [is3 + 2]))
            dsq = rx * rx + ry * ry + rz * rz
            eid = lax.iota(jnp.int32, 16) + v * 16
            c0 = jnp.zeros((16,), jnp.int32)
            plsc.store_scatter(rbuf, [eid, c0], rx)
            plsc.store_scatter(rbuf, [eid, c0 + 1], ry)
            plsc.store_scatter(rbuf, [eid, c0 + 2], rz)
            plsc.store_scatter(rbuf, [eid, c0 + 3], dsq)

    def body(j, carry):
        def halfstep(hbr_c, hbs_c, sem_c, hbr_n, hbs_n, sem_n):
            @pl.when(j + 1 < NCH)
            def _():
                pltpu.async_copy(h_hbm.at[ridx_v.at[j + 1]], hbr_n, sem_n)
                pltpu.async_copy(h_hbm.at[sidx_v.at[j + 1]], hbs_n, sem_n)

            compute_rd(j)
            pltpu.make_async_copy(h_hbm.at[ridx_v.at[j]], hbr_c, sem_c).wait()
            pltpu.make_async_copy(h_hbm.at[sidx_v.at[j]], hbs_c, sem_c).wait()
            o = pl.ds(base + j * CH, CH)
            pltpu.sync_copy(hbr_c, hr_hbm.at[o])
            pltpu.sync_copy(hbs_c, hs_hbm.at[o])
            pltpu.sync_copy(rbuf, rd_hbm.at[o])

        @pl.when(j % 2 == 0)
        def _():
            halfstep(hbr0, hbs0, sem0, hbr1, hbs1, sem1)

        @pl.when(j % 2 == 1)
        def _():
            halfstep(hbr1, hbs1, sem1, hbr0, hbs0, sem0)

        return carry

    lax.fori_loop(0, NCH, body, 0)


@functools.partial(
    pl.kernel,
    out_type=(
        jax.ShapeDtypeStruct((NC, N, D), jnp.float32),
        jax.ShapeDtypeStruct((NC, N, D), jnp.float32),
    ),
    mesh=_mesh,
    scratch_types=[
        pltpu.VMEM((NCH, CH), jnp.int32),
        pltpu.VMEM((CH, D), jnp.float32),
        pltpu.VMEM((CH, D), jnp.float32),
        pltpu.VMEM_SHARED((N, D), jnp.float32),
        pltpu.SemaphoreType.DMA,
        pltpu.SemaphoreType.DMA,
    ],
    compiler_params=pltpu.CompilerParams(needs_layout_passes=False),
)
def _sc_scatter(ridx_hbm, mij_hbm, cd_hbm, z128_hbm,
                macc_hbm, pacc_hbm,
                ridx_v, mbuf0, mbuf1, macc_s, sem0, sem1):
    c = lax.axis_index("c")
    s = lax.axis_index("s")
    w = c * NS + s
    pltpu.sync_copy(ridx_hbm.at[w], ridx_v)
    base = w * EPW

    for src_hbm, out_hbm in ((mij_hbm, macc_hbm), (cd_hbm, pacc_hbm)):
        pltpu.sync_copy(z128_hbm, mbuf0)
        for t in range(8):
            k = s * 8 + t

            @pl.when(k < NZCH)
            def _():
                pltpu.sync_copy(mbuf0, macc_s.at[pl.ds(k * CH, CH)])

        plsc.subcore_barrier()

        def chunk(j):
            return src_hbm.at[pl.ds(base + j * CH, CH)]

        pltpu.async_copy(chunk(0), mbuf0, sem0)

        def body(j, carry):
            @pl.when(j % 2 == 0)
            def _():
                @pl.when(j + 1 < NCH)
                def _():
                    pltpu.async_copy(chunk(j + 1), mbuf1, sem1)

                pltpu.make_async_copy(chunk(j), mbuf0, sem0).wait()
                pltpu.sync_copy(mbuf0, macc_s.at[ridx_v.at[j]], add=True)

            @pl.when(j % 2 == 1)
            def _():
                @pl.when(j + 1 < NCH)
                def _():
                    pltpu.async_copy(chunk(j + 1), mbuf0, sem0)

                pltpu.make_async_copy(chunk(j), mbuf1, sem1).wait()
                pltpu.sync_copy(mbuf1, macc_s.at[ridx_v.at[j]], add=True)

            return carry

        lax.fori_loop(0, NCH, body, 0)
        plsc.subcore_barrier()
        for t in range(8):
            k = s * 8 + t

            @pl.when(k < NZCH)
            def _():
                o = pl.ds(k * CH, CH)
                pltpu.sync_copy(macc_s.at[o], mbuf0)
                pltpu.sync_copy(mbuf0, out_hbm.at[c, o])

        plsc.subcore_barrier()


BE = 1600
GE = E // BE


def _edge_body(hr, hs, e, rd,
               Wr, Ws, We, wd, bm1, Wm2, bm2, Wc1, bc1, Wc2p, bc2p,
               mij_o, cd_o):
    rd_v = rd[...]
    dsq = rd_v[:, 3:4]
    dij = jnp.sqrt(dsq + EPS)
    hid = (jnp.dot(hr[...], Wr[...], preferred_element_type=jnp.float32)
           + jnp.dot(hs[...], Ws[...], preferred_element_type=jnp.float32)
           + jnp.dot(e[...], We[...], preferred_element_type=jnp.float32)
           + dij * wd[...] + bm1[...])
    hid = hid * jax.nn.sigmoid(hid)
    m = jnp.dot(hid, Wm2[...], preferred_element_type=jnp.float32) + bm2[...]
    mij_o[...] = m
    hc = jnp.dot(m, Wc1[...], preferred_element_type=jnp.float32) + bc1[...]
    hc = hc * jax.nn.sigmoid(hc)
    s128 = jnp.dot(hc, Wc2p[...], preferred_element_type=jnp.float32) + bc2p[...]
    lane = lax.broadcasted_iota(jnp.int32, (BE, D), 1)
    rd128 = jnp.concatenate(
        [rd_v, jnp.zeros((BE, D - P16), jnp.float32)], axis=1)
    rd128 = jnp.where(lane < 3, rd128, 0.0)
    cd_o[...] = rd128 * (s128 / (dij + EPS))


def _edge_tc(hr, hs, e, rd, Wr, Ws, We, wd, bm1, Wm2, bm2, Wc1, bc1,
             Wc2p, bc2p):
    full = lambda shape: pl.BlockSpec(shape, lambda i: (0, 0))
    blk = lambda width: pl.BlockSpec((BE, width), lambda i: (i, 0))
    return pl.pallas_call(
        _edge_body,
        grid=(GE,),
        in_specs=[
            blk(D), blk(D), blk(D), blk(P16),
            full((D, 2 * D)), full((D, 2 * D)), full((D, 2 * D)),
            full((1, 2 * D)), full((1, 2 * D)),
            full((2 * D, D)), full((1, D)),
            full((D, D)), full((1, D)),
            full((D, D)), full((1, D)),
        ],
        out_specs=[blk(D), blk(D)],
        out_shape=[
            jax.ShapeDtypeStruct((E, D), jnp.float32),
            jax.ShapeDtypeStruct((E, D), jnp.float32),
        ],
    )(hr, hs, e, rd, Wr, Ws, We, wd, bm1, Wm2, bm2, Wc1, bc1, Wc2p, bc2p)


BN = 2000
GN = N // BN


def _node_body(h, m0, m1, p128, d0, d1,
               Wn1h, Wn1m, bn1, Wn2, bn2, g, b,
               ho_o, po_o):
    m = m0[...] + m1[...]
    hid = (jnp.dot(h[...], Wn1h[...], preferred_element_type=jnp.float32)
           + jnp.dot(m, Wn1m[...], preferred_element_type=jnp.float32)
           + bn1[...])
    hid = hid * jax.nn.sigmoid(hid)
    ho = h[...] + jnp.dot(hid, Wn2[...], preferred_element_type=jnp.float32) + bn2[...]
    mu = jnp.mean(ho, axis=1, keepdims=True)
    zc = ho - mu
    var = jnp.mean(zc * zc, axis=1, keepdims=True)
    ho_o[...] = zc / jnp.sqrt(var + 1e-05) * g[...] + b[...]
    po_o[...] = p128[...] + d0[...] + d1[...]


def _node_tc(h, m0, m1, p128, d0, d1, Wn1h, Wn1m, bn1, Wn2, bn2, g, b):
    full = lambda shape: pl.BlockSpec(shape, lambda i: (0, 0))
    blk = lambda width: pl.BlockSpec((BN, width), lambda i: (i, 0))
    return pl.pallas_call(
        _node_body,
        grid=(GN,),
        in_specs=[
            blk(D), blk(D), blk(D), blk(D), blk(D), blk(D),
            full((D, 2 * D)), full((D, 2 * D)), full((1, 2 * D)),
            full((2 * D, D)), full((1, D)),
            full((1, D)), full((1, D)),
        ],
        out_specs=[blk(D), blk(D)],
        out_shape=[
            jax.ShapeDtypeStruct((N, D), jnp.float32),
            jax.ShapeDtypeStruct((N, D), jnp.float32),
        ],
    )(h, m0, m1, p128, d0, d1, Wn1h, Wn1m, bn1, Wn2, bn2, g, b)


def kernel(h, pos, edge_index, e, Wm1, bm1, Wm2, bm2, Wc1, bc1, Wc2, bc2,
           Wn1, bn1, Wn2, bn2, gamma, beta):
    send = edge_index[0].astype(jnp.int32).reshape(NW, NCH, CH)
    recv = edge_index[1].astype(jnp.int32).reshape(NW, NCH, CH)
    pos128 = jnp.pad(pos, ((0, 0), (0, D - 3)))

    hr, hs, rd = _sc_gather(recv, send, h, jnp.ravel(pos))

    Wr = Wm1[:D]
    Ws = Wm1[D:2 * D]
    We = Wm1[2 * D:3 * D]
    wd = Wm1[3 * D:3 * D + 1]
    Wc2p = jnp.tile(Wc2, (1, D))
    bc2p = jnp.tile(bc2.reshape(1, 1), (1, D))

    mij, cd = _edge_tc(hr, hs, e, rd, Wr, Ws, We, wd,
                       bm1.reshape(1, -1), Wm2, bm2.reshape(1, -1),
                       Wc1, bc1.reshape(1, -1), Wc2p, bc2p)

    z128 = jnp.zeros((CH, D), jnp.float32)
    macc, pacc = _sc_scatter(recv, mij, cd, z128)

    h_out, po128 = _node_tc(h, macc[0], macc[1], pos128, pacc[0], pacc[1],
                            Wn1[:D], Wn1[D:], bn1.reshape(1, -1),
                            Wn2, bn2.reshape(1, -1),
                            gamma.reshape(1, -1), beta.reshape(1, -1))
    return h_out, po128[:, :3]
